# Initial kernel scaffold; baseline (speedup 1.0000x reference)
import jax
import jax.numpy as jnp
from jax.experimental import pallas as pl


def kernel(x, edge_index, batch, W1, W2, W3, W4, L1_W, L1_b, L2_W, L2_b):
    raise NotImplementedError("write your pallas kernel here")


# speedup vs baseline: 4.3478x; 4.3478x over previous
import functools

import jax
import jax.numpy as jnp
from jax import lax
from jax.experimental import pallas as pl
from jax.experimental.pallas import tpu as pltpu
from jax.experimental.pallas import tpu_sc as plsc

N = 100000
G = 128
E = 1600000

NSC = 2
NTILE = 16
NP = 100096
ACC_ROWS = 100224
TILE_N = NP // NTILE
ZTILE = ACC_ROWS // NTILE
POOL_N = NP // (NSC * NTILE)

EB = 128
EP = 1605632
NBLK = EP // EB
BLK_PER_TILE = NBLK // NTILE
BLK_PER_TILE_HALF = NBLK // (2 * NTILE)

_mesh = plsc.VectorSubcoreMesh(core_axis_name="c", subcore_axis_name="s")
_sc_params = pltpu.CompilerParams(use_tc_tiling_on_sc=False,
                                  needs_layout_passes=False)


ZB1 = 2088
ZB2 = 261


def _zero_acc_rows(zbuf, acc, t):
  nb = ZTILE // zbuf.shape[0]
  for j in range(nb):
    pltpu.sync_copy(zbuf, acc.at[pl.ds(t * ZTILE + j * zbuf.shape[0],
                                       zbuf.shape[0])])


def _sc_degree(dst_hbm, zeros_hbm, out_hbm, dstv, onesv, zbuf, stage, acc):
  c = lax.axis_index("c")
  t = lax.axis_index("s")
  pltpu.sync_copy(zeros_hbm, zbuf)
  _zero_acc_rows(zbuf, acc, t)
  for i in range(EB // 16):
    onesv[pl.ds(i * 16, 16)] = jnp.ones((16,), jnp.float32)
  plsc.subcore_barrier()

  base = (c * (NBLK // 2) + t * BLK_PER_TILE_HALF) * EB

  def body(b, carry):
    off = base + b * EB
    pltpu.sync_copy(dst_hbm.at[pl.ds(off, EB)], dstv)
    pltpu.sync_copy(onesv, acc.at[dstv], add=True)
    return carry

  lax.fori_loop(0, BLK_PER_TILE_HALF, body, 0)
  plsc.subcore_barrier()
  for j in range(2):
    sl_a = pl.ds(t * TILE_N + j * (TILE_N // 2), TILE_N // 2)
    sl_o = pl.ds(c * NP + t * TILE_N + j * (TILE_N // 2), TILE_N // 2)
    pltpu.sync_copy(acc.at[sl_a], stage)
    pltpu.sync_copy(stage, out_hbm.at[sl_o])


_deg_kernel = functools.partial(
    pl.kernel,
    out_type=jax.ShapeDtypeStruct((NSC * NP,), jnp.float32),
    mesh=_mesh,
    scratch_types=[
        pltpu.VMEM((EB,), jnp.int32),
        pltpu.VMEM((EB,), jnp.float32),
        pltpu.VMEM((ZB1,), jnp.float32),
        pltpu.VMEM((TILE_N // 2,), jnp.float32),
        pltpu.VMEM_SHARED((ACC_ROWS,), jnp.float32),
    ],
    compiler_params=_sc_params,
)(_sc_degree)


def _make_scatter(nchunk):
  nparts = max(2, nchunk)
  npass = max(1, nchunk // 2)

  def body(g_hbm, src_hbm, dst_hbm, zeros_hbm, out_hbm,
           srcv, dstv, gidxv, rows, zbuf, stage, acc, sem):
    c = lax.axis_index("c")
    t = lax.axis_index("s")
    pltpu.sync_copy(zeros_hbm, zbuf)
    for k in range(npass):
      _zero_acc_rows(zbuf, acc, t)
      plsc.subcore_barrier()
      if nchunk == 1:
        chunk = None
        part = c
        base = (c * (NBLK // 2) + t * BLK_PER_TILE_HALF) * EB
        nblocks = BLK_PER_TILE_HALF
      else:
        chunk = c + 2 * k
        part = chunk
        base = t * BLK_PER_TILE * EB
        nblocks = BLK_PER_TILE

      def ebody(b, carry):
        off = base + b * EB
        pltpu.sync_copy(src_hbm.at[pl.ds(off, EB)], srcv)
        if nchunk == 1:
          idx_ref = srcv
        else:
          for i in range(EB // 16):
            sl = pl.ds(i * 16, 16)
            gidxv[sl] = srcv[sl] * nchunk + chunk
          idx_ref = gidxv
        pltpu.async_copy(g_hbm.at[idx_ref], rows, sem).wait()
        pltpu.sync_copy(dst_hbm.at[pl.ds(off, EB)], dstv)
        pltpu.sync_copy(rows, acc.at[dstv], add=True)
        return carry

      lax.fori_loop(0, nblocks, ebody, 0)
      plsc.subcore_barrier()
      for j in range(8):
        sl = pl.ds(t * TILE_N + j * (TILE_N // 8), TILE_N // 8)
        pltpu.sync_copy(acc.at[sl], stage)
        pltpu.sync_copy(stage, out_hbm.at[part, sl])
      plsc.subcore_barrier()

  return pl.kernel(
      body,
      out_type=jax.ShapeDtypeStruct((nparts, NP, 16), jnp.float32),
      mesh=_mesh,
      scratch_types=[
          pltpu.VMEM((EB,), jnp.int32),
          pltpu.VMEM((EB,), jnp.int32),
          pltpu.VMEM((EB,), jnp.int32),
          pltpu.VMEM((EB, 16), jnp.float32),
          pltpu.VMEM((ZB2, 16), jnp.float32),
          pltpu.VMEM((TILE_N // 8, 16), jnp.float32),
          pltpu.VMEM_SHARED((ACC_ROWS, 16), jnp.float32),
          pltpu.SemaphoreType.DMA,
      ],
      compiler_params=_sc_params,
  )


_scatter1 = _make_scatter(1)
_scatter2 = _make_scatter(2)
_scatter4 = _make_scatter(4)
_scatter8 = _make_scatter(8)


_POOL_SUB = 512


def _sc_pool(x5_hbm, batch_hbm, out_hbm, xv, bv, table):
  c = lax.axis_index("c")
  t = lax.axis_index("s")
  wid = t * NSC + c
  base = wid * POOL_N
  iota = lax.iota(jnp.int32, 16)
  neginf = jnp.full((16,), -jnp.inf, jnp.float32)

  def init(i, carry):
    plsc.store_scatter(table, [i * 16 + iota], neginf)
    return carry

  lax.fori_loop(0, (G * G) // 16, init, 0)

  nsub = -(-POOL_N // _POOL_SUB)
  for j in range(nsub):
    size = min(_POOL_SUB, POOL_N - j * _POOL_SUB)
    off = base + j * _POOL_SUB
    pltpu.sync_copy(x5_hbm.at[pl.ds(off * 128, size * 128)],
                    xv.at[pl.ds(0, size * 128)])
    pltpu.sync_copy(batch_hbm.at[pl.ds(off, size)], bv.at[pl.ds(0, size)])

    def rbody(r, carry):
      ridx = jnp.full((16,), r, jnp.int32)
      bid = plsc.load_gather(bv, [ridx])
      for c8 in range(8):
        xidx = r * 128 + c8 * 16 + iota
        tidx = bid * G + c8 * 16 + iota
        val = plsc.load_gather(xv, [xidx])
        cur = plsc.load_gather(table, [tidx])
        plsc.store_scatter(table, [tidx], jnp.maximum(cur, val))
      return carry

    lax.fori_loop(0, size, rbody, 0)

  pltpu.sync_copy(table, out_hbm.at[wid])


_pool_kernel = functools.partial(
    pl.kernel,
    out_type=jax.ShapeDtypeStruct((NSC * NTILE, G * G), jnp.float32),
    mesh=_mesh,
    scratch_types=[
        pltpu.VMEM((_POOL_SUB * 128,), jnp.float32),
        pltpu.VMEM((_POOL_SUB,), jnp.int32),
        pltpu.VMEM((G * G,), jnp.float32),
    ],
    compiler_params=_sc_params,
)(_sc_pool)


BN = 3128
TC_GRID = NP // BN


def _l2n(x):
  return x / (jnp.sqrt(jnp.sum(x * x, axis=1, keepdims=True)) + 1e-8)


def _tc1_body(x_ref, deg_ref, w_ref, g_ref, dinv_ref):
  deg = deg_ref[0] + deg_ref[1] + 1.0
  dinv = lax.rsqrt(deg)
  g = jnp.dot(_l2n(x_ref[...]), w_ref[...],
              preferred_element_type=jnp.float32) * dinv
  g_ref[...] = g
  dinv_ref[...] = dinv


def _tc1(xp, deg_parts, w1p):
  return pl.pallas_call(
      _tc1_body,
      grid=(TC_GRID,),
      in_specs=[
          pl.BlockSpec((BN, 8), lambda i: (i, 0)),
          pl.BlockSpec((2, BN, 1), lambda i: (0, i, 0)),
          pl.BlockSpec((8, 16), lambda i: (0, 0)),
      ],
      out_specs=[
          pl.BlockSpec((BN, 16), lambda i: (i, 0)),
          pl.BlockSpec((BN, 1), lambda i: (i, 0)),
      ],
      out_shape=[
          jax.ShapeDtypeStruct((NP, 16), jnp.float32),
          jax.ShapeDtypeStruct((NP, 1), jnp.float32),
      ],
  )(xp, deg_parts, w1p)


def _make_tc_layer(P, d_prev, d_out, sum_parts):
  def body(parts_ref, g_ref, dinv_ref, w_ref, out_ref):
    if sum_parts:
      agg = parts_ref[0] + parts_ref[1]
    else:
      agg = jnp.concatenate([parts_ref[p] for p in range(P)], axis=1)
    dinv = dinv_ref[...]
    x = jax.nn.relu(dinv * (agg + g_ref[...]))
    out_ref[...] = jnp.dot(_l2n(x), w_ref[...],
                           preferred_element_type=jnp.float32) * dinv

  def run(parts, g_prev, dinv, w):
    return pl.pallas_call(
        body,
        grid=(TC_GRID,),
        in_specs=[
            pl.BlockSpec((P, BN, 16), lambda i: (0, i, 0)),
            pl.BlockSpec((BN, d_prev), lambda i: (i, 0)),
            pl.BlockSpec((BN, 1), lambda i: (i, 0)),
            pl.BlockSpec((d_prev, d_out), lambda i: (0, 0)),
        ],
        out_specs=pl.BlockSpec((BN, d_out), lambda i: (i, 0)),
        out_shape=jax.ShapeDtypeStruct((NP, d_out), jnp.float32),
    )(parts, g_prev, dinv, w)

  return run


_tc_layer2 = _make_tc_layer(2, 16, 32, True)
_tc_layer3 = _make_tc_layer(2, 32, 64, False)
_tc_layer4 = _make_tc_layer(4, 64, 128, False)


def _tc_x5_body(parts_ref, g_ref, dinv_ref, out_ref):
  agg = jnp.concatenate([parts_ref[p] for p in range(8)], axis=1)
  out_ref[...] = jax.nn.relu(dinv_ref[...] * (agg + g_ref[...]))


def _tc_x5(parts, g4, dinv):
  return pl.pallas_call(
      _tc_x5_body,
      grid=(TC_GRID,),
      in_specs=[
          pl.BlockSpec((8, BN, 16), lambda i: (0, i, 0)),
          pl.BlockSpec((BN, 128), lambda i: (i, 0)),
          pl.BlockSpec((BN, 1), lambda i: (i, 0)),
      ],
      out_specs=pl.BlockSpec((BN, 128), lambda i: (i, 0)),
      out_shape=jax.ShapeDtypeStruct((NP, 128), jnp.float32),
  )(parts, g4, dinv)


def _tc_heads_body(tab_ref, w1_ref, b1_ref, w2_ref, b2_ref, out_ref):
  pooled = jnp.max(tab_ref[...], axis=0)
  pooled = jnp.maximum(pooled, 0.0)
  h = jax.nn.relu(jnp.dot(_l2n(pooled), w1_ref[...],
                          preferred_element_type=jnp.float32) + b1_ref[...])
  out_ref[...] = jax.nn.relu(jnp.dot(_l2n(h), w2_ref[...],
                                     preferred_element_type=jnp.float32)
                             + b2_ref[...])


def _tc_heads(tables, l1w, l1b, l2wp, l2bp):
  return pl.pallas_call(
      _tc_heads_body,
      out_shape=jax.ShapeDtypeStruct((G, 16), jnp.float32),
  )(tables, l1w, l1b, l2wp, l2bp)


def kernel(x, edge_index, batch, W1, W2, W3, W4, L1_W, L1_b, L2_W, L2_b):
  f32 = jnp.float32
  src = edge_index[0]
  dst = edge_index[1]

  pad = EP - E
  pad_src = (jnp.arange(pad, dtype=jnp.int32) % 1024)
  pad_dst = NP + (jnp.arange(pad, dtype=jnp.int32) % 64)
  src_p = jnp.concatenate([src, pad_src])
  dst_p = jnp.concatenate([dst, pad_dst])

  xp = jnp.zeros((NP, 8), f32).at[:N, :2].set(x)
  batch_p = jnp.concatenate(
      [batch, jnp.full((NP - N,), G - 1, jnp.int32)])
  w1p = jnp.zeros((8, 16), f32).at[:2].set(W1)

  zeros2d = jnp.zeros((ZB2, 16), f32)
  zeros1d = jnp.zeros((ZB1,), f32)

  deg_parts = _deg_kernel(dst_p, zeros1d)

  g1, dinv = _tc1(xp, deg_parts.reshape(NSC, NP, 1), w1p)

  parts1 = _scatter1(g1, src_p, dst_p, zeros2d)
  g2 = _tc_layer2(parts1, g1, dinv, W2)

  parts2 = _scatter2(g2.reshape(NP * 2, 16), src_p, dst_p, zeros2d)
  g3 = _tc_layer3(parts2, g2, dinv, W3)

  parts3 = _scatter4(g3.reshape(NP * 4, 16), src_p, dst_p, zeros2d)
  g4 = _tc_layer4(parts3, g3, dinv, W4)

  parts4 = _scatter8(g4.reshape(NP * 8, 16), src_p, dst_p, zeros2d)
  x5 = _tc_x5(parts4, g4, dinv)

  tables = _pool_kernel(x5.reshape(NP * 128), batch_p)

  l2wp = jnp.zeros((64, 16), f32).at[:, :10].set(L2_W)
  l2bp = jnp.zeros((16,), f32).at[:10].set(L2_b)
  out = _tc_heads(tables.reshape(NSC * NTILE, G, G),
                  L1_W, L1_b.reshape(1, 64), l2wp, l2bp.reshape(1, 16))
  return out[:, :10]


# speedup vs baseline: 9.0132x; 2.0731x over previous
import functools

import jax
import jax.numpy as jnp
from jax import lax
from jax.experimental import pallas as pl
from jax.experimental.pallas import tpu as pltpu
from jax.experimental.pallas import tpu_sc as plsc

N = 100000
G = 128
E = 1600000

NSC = 2
NTILE = 16
NP = 100096
ACC_ROWS = 100224
TILE_N = NP // NTILE
ZTILE = ACC_ROWS // NTILE
POOL_N = NP // (NSC * NTILE)

EB = 128
EP = 1605632
NBLK = EP // EB
BLK_PER_TILE = NBLK // NTILE
BLK_PER_TILE_HALF = NBLK // (2 * NTILE)

_mesh = plsc.VectorSubcoreMesh(core_axis_name="c", subcore_axis_name="s")
_sc_params = pltpu.CompilerParams(use_tc_tiling_on_sc=False,
                                  needs_layout_passes=False)


ZB1 = 2088
ZB2 = 261


def _zero_acc_rows(zbuf, acc, t):
  nb = ZTILE // zbuf.shape[0]
  for j in range(nb):
    pltpu.sync_copy(zbuf, acc.at[pl.ds(t * ZTILE + j * zbuf.shape[0],
                                       zbuf.shape[0])])


def _sc_degree(dst_hbm, zeros_hbm, out_hbm, dstv, onesv, zbuf, stage, acc):
  c = lax.axis_index("c")
  t = lax.axis_index("s")
  pltpu.sync_copy(zeros_hbm, zbuf)
  _zero_acc_rows(zbuf, acc, t)
  for i in range(EB // 16):
    onesv[pl.ds(i * 16, 16)] = jnp.ones((16,), jnp.float32)
  plsc.subcore_barrier()

  base = (c * (NBLK // 2) + t * BLK_PER_TILE_HALF) * EB

  def body(b, carry):
    off = base + b * EB
    pltpu.sync_copy(dst_hbm.at[pl.ds(off, EB)], dstv)
    pltpu.sync_copy(onesv, acc.at[dstv], add=True)
    return carry

  lax.fori_loop(0, BLK_PER_TILE_HALF, body, 0)
  plsc.subcore_barrier()
  for j in range(2):
    sl_a = pl.ds(t * TILE_N + j * (TILE_N // 2), TILE_N // 2)
    sl_o = pl.ds(c * NP + t * TILE_N + j * (TILE_N // 2), TILE_N // 2)
    pltpu.sync_copy(acc.at[sl_a], stage)
    pltpu.sync_copy(stage, out_hbm.at[sl_o])


_deg_kernel = functools.partial(
    pl.kernel,
    out_type=jax.ShapeDtypeStruct((NSC * NP,), jnp.float32),
    mesh=_mesh,
    scratch_types=[
        pltpu.VMEM((EB,), jnp.int32),
        pltpu.VMEM((EB,), jnp.float32),
        pltpu.VMEM((ZB1,), jnp.float32),
        pltpu.VMEM((TILE_N // 2,), jnp.float32),
        pltpu.VMEM_SHARED((ACC_ROWS,), jnp.float32),
    ],
    compiler_params=_sc_params,
)(_sc_degree)


def _make_scatter(nchunk):
  nparts = max(2, nchunk)
  npass = max(1, nchunk // 2)

  def body(g_hbm, src_hbm, dst_hbm, zeros_hbm, out_hbm,
           srcv0, dstv0, gidx0, rows0, srcv1, dstv1, gidx1, rows1,
           zbuf, stage, acc, semE0, semE1, semG0, semG1):
    c = lax.axis_index("c")
    t = lax.axis_index("s")
    sets = ((srcv0, dstv0, gidx0, rows0, semE0, semG0),
            (srcv1, dstv1, gidx1, rows1, semE1, semG1))

    def start_e(off, st):
      srcv, dstv, _, _, semE, _ = st
      pltpu.async_copy(src_hbm.at[pl.ds(off, EB)], srcv, semE)
      pltpu.async_copy(dst_hbm.at[pl.ds(off, EB)], dstv, semE)

    def wait_e(st):
      srcv, dstv, _, _, semE, _ = st
      pltpu.make_async_copy(src_hbm.at[pl.ds(0, EB)], srcv, semE).wait()
      pltpu.make_async_copy(dst_hbm.at[pl.ds(0, EB)], dstv, semE).wait()

    def start_g(st, chunk):
      srcv, _, gidxv, rows, _, semG = st
      if nchunk == 1:
        idx_ref = srcv
      else:
        for i in range(EB // 16):
          sl = pl.ds(i * 16, 16)
          gidxv[sl] = srcv[sl] * nchunk + chunk
        idx_ref = gidxv
      pltpu.async_copy(g_hbm.at[idx_ref], rows, semG)

    def drain_scatter(st):
      _, dstv, _, rows, _, semG = st
      pltpu.make_async_copy(g_hbm.at[dstv], rows, semG).wait()
      pltpu.sync_copy(rows, acc.at[dstv], add=True)

    pltpu.sync_copy(zeros_hbm, zbuf)
    for k in range(npass):
      _zero_acc_rows(zbuf, acc, t)
      plsc.subcore_barrier()
      if nchunk == 1:
        chunk = None
        part = c
        base = (c * (NBLK // 2) + t * BLK_PER_TILE_HALF) * EB
        nblocks = BLK_PER_TILE_HALF
      else:
        chunk = c + 2 * k
        part = chunk
        base = t * BLK_PER_TILE * EB
        nblocks = BLK_PER_TILE

      start_e(base, sets[0])

      def ebody(s, carry):
        for ph in range(2):
          b = 2 * s + ph
          cur, nxt = sets[ph], sets[1 - ph]
          wait_e(cur)
          start_g(cur, chunk)
          if ph == 0:
            @pl.when(s > 0)
            def _():
              drain_scatter(nxt)
          else:
            drain_scatter(nxt)
          start_e(base + (b + 1) * EB, nxt)
        return carry

      lax.fori_loop(0, nblocks // 2, ebody, 0)
      drain_scatter(sets[1])
      wait_e(sets[0])
      plsc.subcore_barrier()
      for j in range(16):
        sl = pl.ds(t * TILE_N + j * (TILE_N // 16), TILE_N // 16)
        pltpu.sync_copy(acc.at[sl], stage)
        pltpu.sync_copy(stage, out_hbm.at[part, sl])
      plsc.subcore_barrier()

  return pl.kernel(
      body,
      out_type=jax.ShapeDtypeStruct((nparts, NP, 16), jnp.float32),
      mesh=_mesh,
      scratch_types=[
          pltpu.VMEM((EB,), jnp.int32),
          pltpu.VMEM((EB,), jnp.int32),
          pltpu.VMEM((EB,), jnp.int32),
          pltpu.VMEM((EB, 16), jnp.float32),
          pltpu.VMEM((EB,), jnp.int32),
          pltpu.VMEM((EB,), jnp.int32),
          pltpu.VMEM((EB,), jnp.int32),
          pltpu.VMEM((EB, 16), jnp.float32),
          pltpu.VMEM((ZB2, 16), jnp.float32),
          pltpu.VMEM((TILE_N // 16, 16), jnp.float32),
          pltpu.VMEM_SHARED((ACC_ROWS, 16), jnp.float32),
          pltpu.SemaphoreType.DMA,
          pltpu.SemaphoreType.DMA,
          pltpu.SemaphoreType.DMA,
          pltpu.SemaphoreType.DMA,
      ],
      compiler_params=_sc_params,
  )


_scatter1 = _make_scatter(1)
_scatter2 = _make_scatter(2)
_scatter4 = _make_scatter(4)
_scatter8 = _make_scatter(8)


_POOL_SUB = 512


def _sc_pool(x5_hbm, batch_hbm, out_hbm, xv, bv, table):
  c = lax.axis_index("c")
  t = lax.axis_index("s")
  wid = t * NSC + c
  base = wid * POOL_N
  iota = lax.iota(jnp.int32, 16)
  neginf = jnp.full((16,), -jnp.inf, jnp.float32)

  def init(i, carry):
    plsc.store_scatter(table, [i * 16 + iota], neginf)
    return carry

  lax.fori_loop(0, (G * G) // 16, init, 0)

  nsub = -(-POOL_N // _POOL_SUB)
  for j in range(nsub):
    size = min(_POOL_SUB, POOL_N - j * _POOL_SUB)
    off = base + j * _POOL_SUB
    pltpu.sync_copy(x5_hbm.at[pl.ds(off * 128, size * 128)],
                    xv.at[pl.ds(0, size * 128)])
    pltpu.sync_copy(batch_hbm.at[pl.ds(off, size)], bv.at[pl.ds(0, size)])

    def rbody(r, carry):
      ridx = jnp.full((16,), r, jnp.int32)
      bid = plsc.load_gather(bv, [ridx])
      for c8 in range(8):
        xidx = r * 128 + c8 * 16 + iota
        tidx = bid * G + c8 * 16 + iota
        val = plsc.load_gather(xv, [xidx])
        cur = plsc.load_gather(table, [tidx])
        plsc.store_scatter(table, [tidx], jnp.maximum(cur, val))
      return carry

    lax.fori_loop(0, size, rbody, 0)

  pltpu.sync_copy(table, out_hbm.at[wid])


_pool_kernel = functools.partial(
    pl.kernel,
    out_type=jax.ShapeDtypeStruct((NSC * NTILE, G * G), jnp.float32),
    mesh=_mesh,
    scratch_types=[
        pltpu.VMEM((_POOL_SUB * 128,), jnp.float32),
        pltpu.VMEM((_POOL_SUB,), jnp.int32),
        pltpu.VMEM((G * G,), jnp.float32),
    ],
    compiler_params=_sc_params,
)(_sc_pool)


BN = 3128
TC_GRID = NP // BN


def _l2n(x):
  return x / (jnp.sqrt(jnp.sum(x * x, axis=1, keepdims=True)) + 1e-8)


def _tc1_body(x_ref, deg_ref, w_ref, g_ref, dinv_ref):
  deg = deg_ref[0] + deg_ref[1] + 1.0
  dinv = lax.rsqrt(deg)
  g = jnp.dot(_l2n(x_ref[...]), w_ref[...],
              preferred_element_type=jnp.float32) * dinv
  g_ref[...] = g
  dinv_ref[...] = dinv


def _tc1(xp, deg_parts, w1p):
  return pl.pallas_call(
      _tc1_body,
      grid=(TC_GRID,),
      in_specs=[
          pl.BlockSpec((BN, 8), lambda i: (i, 0)),
          pl.BlockSpec((2, BN, 1), lambda i: (0, i, 0)),
          pl.BlockSpec((8, 16), lambda i: (0, 0)),
      ],
      out_specs=[
          pl.BlockSpec((BN, 16), lambda i: (i, 0)),
          pl.BlockSpec((BN, 1), lambda i: (i, 0)),
      ],
      out_shape=[
          jax.ShapeDtypeStruct((NP, 16), jnp.float32),
          jax.ShapeDtypeStruct((NP, 1), jnp.float32),
      ],
  )(xp, deg_parts, w1p)


def _make_tc_layer(P, d_prev, d_out, sum_parts):
  def body(parts_ref, g_ref, dinv_ref, w_ref, out_ref):
    if sum_parts:
      agg = parts_ref[0] + parts_ref[1]
    else:
      agg = jnp.concatenate([parts_ref[p] for p in range(P)], axis=1)
    dinv = dinv_ref[...]
    x = jax.nn.relu(dinv * (agg + g_ref[...]))
    out_ref[...] = jnp.dot(_l2n(x), w_ref[...],
                           preferred_element_type=jnp.float32) * dinv

  def run(parts, g_prev, dinv, w):
    return pl.pallas_call(
        body,
        grid=(TC_GRID,),
        in_specs=[
            pl.BlockSpec((P, BN, 16), lambda i: (0, i, 0)),
            pl.BlockSpec((BN, d_prev), lambda i: (i, 0)),
            pl.BlockSpec((BN, 1), lambda i: (i, 0)),
            pl.BlockSpec((d_prev, d_out), lambda i: (0, 0)),
        ],
        out_specs=pl.BlockSpec((BN, d_out), lambda i: (i, 0)),
        out_shape=jax.ShapeDtypeStruct((NP, d_out), jnp.float32),
    )(parts, g_prev, dinv, w)

  return run


_tc_layer2 = _make_tc_layer(2, 16, 32, True)
_tc_layer3 = _make_tc_layer(2, 32, 64, False)
_tc_layer4 = _make_tc_layer(4, 64, 128, False)


def _tc_x5_body(parts_ref, g_ref, dinv_ref, out_ref):
  agg = jnp.concatenate([parts_ref[p] for p in range(8)], axis=1)
  out_ref[...] = jax.nn.relu(dinv_ref[...] * (agg + g_ref[...]))


def _tc_x5(parts, g4, dinv):
  return pl.pallas_call(
      _tc_x5_body,
      grid=(TC_GRID,),
      in_specs=[
          pl.BlockSpec((8, BN, 16), lambda i: (0, i, 0)),
          pl.BlockSpec((BN, 128), lambda i: (i, 0)),
          pl.BlockSpec((BN, 1), lambda i: (i, 0)),
      ],
      out_specs=pl.BlockSpec((BN, 128), lambda i: (i, 0)),
      out_shape=jax.ShapeDtypeStruct((NP, 128), jnp.float32),
  )(parts, g4, dinv)


def _tc_heads_body(tab_ref, w1_ref, b1_ref, w2_ref, b2_ref, out_ref):
  pooled = jnp.max(tab_ref[...], axis=0)
  pooled = jnp.maximum(pooled, 0.0)
  h = jax.nn.relu(jnp.dot(_l2n(pooled), w1_ref[...],
                          preferred_element_type=jnp.float32) + b1_ref[...])
  out_ref[...] = jax.nn.relu(jnp.dot(_l2n(h), w2_ref[...],
                                     preferred_element_type=jnp.float32)
                             + b2_ref[...])


def _tc_heads(tables, l1w, l1b, l2wp, l2bp):
  return pl.pallas_call(
      _tc_heads_body,
      out_shape=jax.ShapeDtypeStruct((G, 16), jnp.float32),
  )(tables, l1w, l1b, l2wp, l2bp)


def kernel(x, edge_index, batch, W1, W2, W3, W4, L1_W, L1_b, L2_W, L2_b):
  f32 = jnp.float32
  src = edge_index[0]
  dst = edge_index[1]

  pad = EP + EB - E
  pad_src = (jnp.arange(pad, dtype=jnp.int32) % 1024)
  pad_dst = NP + (jnp.arange(pad, dtype=jnp.int32) % 64)
  src_p = jnp.concatenate([src, pad_src])
  dst_p = jnp.concatenate([dst, pad_dst])

  xp = jnp.zeros((NP, 8), f32).at[:N, :2].set(x)
  batch_p = jnp.concatenate(
      [batch, jnp.full((NP - N,), G - 1, jnp.int32)])
  w1p = jnp.zeros((8, 16), f32).at[:2].set(W1)

  zeros2d = jnp.zeros((ZB2, 16), f32)
  zeros1d = jnp.zeros((ZB1,), f32)

  deg_parts = _deg_kernel(dst_p, zeros1d)

  g1, dinv = _tc1(xp, deg_parts.reshape(NSC, NP, 1), w1p)

  parts1 = _scatter1(g1, src_p, dst_p, zeros2d)
  g2 = _tc_layer2(parts1, g1, dinv, W2)

  parts2 = _scatter2(g2.reshape(NP * 2, 16), src_p, dst_p, zeros2d)
  g3 = _tc_layer3(parts2, g2, dinv, W3)

  parts3 = _scatter4(g3.reshape(NP * 4, 16), src_p, dst_p, zeros2d)
  g4 = _tc_layer4(parts3, g3, dinv, W4)

  parts4 = _scatter8(g4.reshape(NP * 8, 16), src_p, dst_p, zeros2d)
  x5 = _tc_x5(parts4, g4, dinv)

  tables = _pool_kernel(x5.reshape(NP * 128), batch_p)

  l2wp = jnp.zeros((64, 16), f32).at[:, :10].set(L2_W)
  l2bp = jnp.zeros((16,), f32).at[:10].set(L2_b)
  out = _tc_heads(tables.reshape(NSC * NTILE, G, G),
                  L1_W, L1_b.reshape(1, 64), l2wp, l2bp.reshape(1, 16))
  return out[:, :10]


# speedup vs baseline: 14.6184x; 1.6219x over previous
import functools

import jax
import jax.numpy as jnp
from jax import lax
from jax.experimental import pallas as pl
from jax.experimental.pallas import tpu as pltpu
from jax.experimental.pallas import tpu_sc as plsc

N = 100000
G = 128
E = 1600000

NSC = 2
NTILE = 16
NP = 100096
ACC_ROWS = 100224
TILE_N = NP // NTILE
ZTILE = ACC_ROWS // NTILE
POOL_N = NP // (NSC * NTILE)

EB = 128
EP = 1605632
NBLK = EP // EB
BLK_PER_TILE = NBLK // NTILE
BLK_PER_TILE_HALF = NBLK // (2 * NTILE)

_mesh = plsc.VectorSubcoreMesh(core_axis_name="c", subcore_axis_name="s")
_sc_params = pltpu.CompilerParams(use_tc_tiling_on_sc=False,
                                  needs_layout_passes=False)


ZB1 = 2088
ZB2 = 261


def _zero_acc_rows(zbuf, acc, t):
  nb = ZTILE // zbuf.shape[0]
  for j in range(nb):
    pltpu.sync_copy(zbuf, acc.at[pl.ds(t * ZTILE + j * zbuf.shape[0],
                                       zbuf.shape[0])])


def _sc_degree(dst_hbm, zeros_hbm, out_hbm, dstv, onesv, zbuf, stage, acc):
  c = lax.axis_index("c")
  t = lax.axis_index("s")
  pltpu.sync_copy(zeros_hbm, zbuf)
  _zero_acc_rows(zbuf, acc, t)
  for i in range(EB // 16):
    onesv[pl.ds(i * 16, 16)] = jnp.ones((16,), jnp.float32)
  plsc.subcore_barrier()

  base = (c * (NBLK // 2) + t * BLK_PER_TILE_HALF) * EB

  def body(b, carry):
    off = base + b * EB
    pltpu.sync_copy(dst_hbm.at[pl.ds(off, EB)], dstv)
    pltpu.sync_copy(onesv, acc.at[dstv], add=True)
    return carry

  lax.fori_loop(0, BLK_PER_TILE_HALF, body, 0)
  plsc.subcore_barrier()
  for j in range(2):
    sl_a = pl.ds(t * TILE_N + j * (TILE_N // 2), TILE_N // 2)
    sl_o = pl.ds(c * NP + t * TILE_N + j * (TILE_N // 2), TILE_N // 2)
    pltpu.sync_copy(acc.at[sl_a], stage)
    pltpu.sync_copy(stage, out_hbm.at[sl_o])


_deg_kernel = functools.partial(
    pl.kernel,
    out_type=jax.ShapeDtypeStruct((NSC * NP,), jnp.float32),
    mesh=_mesh,
    scratch_types=[
        pltpu.VMEM((EB,), jnp.int32),
        pltpu.VMEM((EB,), jnp.float32),
        pltpu.VMEM((ZB1,), jnp.float32),
        pltpu.VMEM((TILE_N // 2,), jnp.float32),
        pltpu.VMEM_SHARED((ACC_ROWS,), jnp.float32),
    ],
    compiler_params=_sc_params,
)(_sc_degree)


def _make_scatter(nchunk):
  nparts = max(2, nchunk)
  npass = max(1, nchunk // 2)

  NE = 8
  NR = 4

  def body(g_hbm, src_hbm, dst_hbm, zeros_hbm, out_hbm, *scr):
    ebufs = [scr[3 * i:3 * i + 3] for i in range(NE)]
    rbufs = [scr[3 * NE + 3 * i:3 * NE + 3 * i + 3] for i in range(NR)]
    zbuf, stage, acc = scr[3 * NE + 3 * NR:]
    c = lax.axis_index("c")
    t = lax.axis_index("s")

    def start_e(off, eb):
      srcv, dstv, semE = eb
      pltpu.async_copy(src_hbm.at[pl.ds(off, EB)], srcv, semE)
      pltpu.async_copy(dst_hbm.at[pl.ds(off, EB)], dstv, semE)

    def wait_e(eb):
      srcv, dstv, semE = eb
      pltpu.make_async_copy(src_hbm.at[pl.ds(0, EB)], srcv, semE).wait()
      pltpu.make_async_copy(dst_hbm.at[pl.ds(0, EB)], dstv, semE).wait()

    def start_g(eb, rb, chunk):
      srcv = eb[0]
      rows, gidxv, semG = rb
      if nchunk == 1:
        idx_ref = srcv
      else:
        for i in range(EB // 16):
          sl = pl.ds(i * 16, 16)
          gidxv[sl] = srcv[sl] * nchunk + chunk
        idx_ref = gidxv
      pltpu.async_copy(g_hbm.at[idx_ref], rows, semG)

    def drain_scatter(eb, rb):
      dstv = eb[1]
      rows, _, semG = rb
      pltpu.make_async_copy(g_hbm.at[dstv], rows, semG).wait()
      pltpu.sync_copy(rows, acc.at[dstv], add=True)

    pltpu.sync_copy(zeros_hbm, zbuf)
    for k in range(npass):
      _zero_acc_rows(zbuf, acc, t)
      plsc.subcore_barrier()
      if nchunk == 1:
        chunk = None
        part = c
        base = (c * (NBLK // 2) + t * BLK_PER_TILE_HALF) * EB
        nblocks = BLK_PER_TILE_HALF
      else:
        chunk = c + 2 * k
        part = chunk
        base = t * BLK_PER_TILE * EB
        nblocks = BLK_PER_TILE

      for j in range(4):
        start_e(base + j * EB, ebufs[j])

      def ebody(s, carry):
        for ph in range(NE):
          b = NE * s + ph
          wait_e(ebufs[ph])
          start_g(ebufs[ph], rbufs[ph % NR], chunk)
          if ph < 2:
            @pl.when(s > 0)
            def _():
              drain_scatter(ebufs[(ph - 2) % NE], rbufs[(ph - 2) % NR])
          else:
            drain_scatter(ebufs[ph - 2], rbufs[(ph - 2) % NR])
          start_e(base + (b + 4) * EB, ebufs[(ph + 4) % NE])
        return carry

      lax.fori_loop(0, nblocks // NE, ebody, 0)
      drain_scatter(ebufs[NE - 2], rbufs[(NE - 2) % NR])
      drain_scatter(ebufs[NE - 1], rbufs[(NE - 1) % NR])
      for j in range(4):
        wait_e(ebufs[j])
      plsc.subcore_barrier()
      for j in range(16):
        sl = pl.ds(t * TILE_N + j * (TILE_N // 16), TILE_N // 16)
        pltpu.sync_copy(acc.at[sl], stage)
        pltpu.sync_copy(stage, out_hbm.at[part, sl])
      plsc.subcore_barrier()

  escr = []
  for _ in range(NE):
    escr += [pltpu.VMEM((EB,), jnp.int32), pltpu.VMEM((EB,), jnp.int32),
             pltpu.SemaphoreType.DMA]
  for _ in range(NR):
    escr += [pltpu.VMEM((EB, 16), jnp.float32), pltpu.VMEM((EB,), jnp.int32),
             pltpu.SemaphoreType.DMA]
  escr += [
      pltpu.VMEM((ZB2, 16), jnp.float32),
      pltpu.VMEM((TILE_N // 16, 16), jnp.float32),
      pltpu.VMEM_SHARED((ACC_ROWS, 16), jnp.float32),
  ]
  return pl.kernel(
      body,
      out_type=jax.ShapeDtypeStruct((nparts, NP, 16), jnp.float32),
      mesh=_mesh,
      scratch_types=escr,
      compiler_params=_sc_params,
  )


_scatter1 = _make_scatter(1)
_scatter2 = _make_scatter(2)
_scatter4 = _make_scatter(4)
_scatter8 = _make_scatter(8)


_POOL_SUB = 512


def _sc_pool(x5_hbm, batch_hbm, out_hbm, xv, bv, table):
  c = lax.axis_index("c")
  t = lax.axis_index("s")
  wid = t * NSC + c
  base = wid * POOL_N
  iota = lax.iota(jnp.int32, 16)
  neginf = jnp.full((16,), -jnp.inf, jnp.float32)

  def init(i, carry):
    plsc.store_scatter(table, [i * 16 + iota], neginf)
    return carry

  lax.fori_loop(0, (G * G) // 16, init, 0)

  nsub = -(-POOL_N // _POOL_SUB)
  for j in range(nsub):
    size = min(_POOL_SUB, POOL_N - j * _POOL_SUB)
    off = base + j * _POOL_SUB
    pltpu.sync_copy(x5_hbm.at[pl.ds(off * 128, size * 128)],
                    xv.at[pl.ds(0, size * 128)])
    pltpu.sync_copy(batch_hbm.at[pl.ds(off, size)], bv.at[pl.ds(0, size)])

    def rbody(r, carry):
      ridx = jnp.full((16,), r, jnp.int32)
      bid = plsc.load_gather(bv, [ridx])
      for c8 in range(8):
        xidx = r * 128 + c8 * 16 + iota
        tidx = bid * G + c8 * 16 + iota
        val = plsc.load_gather(xv, [xidx])
        cur = plsc.load_gather(table, [tidx])
        plsc.store_scatter(table, [tidx], jnp.maximum(cur, val))
      return carry

    lax.fori_loop(0, size, rbody, 0)

  pltpu.sync_copy(table, out_hbm.at[wid])


_pool_kernel = functools.partial(
    pl.kernel,
    out_type=jax.ShapeDtypeStruct((NSC * NTILE, G * G), jnp.float32),
    mesh=_mesh,
    scratch_types=[
        pltpu.VMEM((_POOL_SUB * 128,), jnp.float32),
        pltpu.VMEM((_POOL_SUB,), jnp.int32),
        pltpu.VMEM((G * G,), jnp.float32),
    ],
    compiler_params=_sc_params,
)(_sc_pool)


BN = 3128
TC_GRID = NP // BN


def _l2n(x):
  return x / (jnp.sqrt(jnp.sum(x * x, axis=1, keepdims=True)) + 1e-8)


def _tc1_body(x_ref, deg_ref, w_ref, g_ref, dinv_ref):
  deg = deg_ref[0] + deg_ref[1] + 1.0
  dinv = lax.rsqrt(deg)
  g = jnp.dot(_l2n(x_ref[...]), w_ref[...],
              preferred_element_type=jnp.float32) * dinv
  g_ref[...] = g
  dinv_ref[...] = dinv


def _tc1(xp, deg_parts, w1p):
  return pl.pallas_call(
      _tc1_body,
      grid=(TC_GRID,),
      in_specs=[
          pl.BlockSpec((BN, 8), lambda i: (i, 0)),
          pl.BlockSpec((2, BN, 1), lambda i: (0, i, 0)),
          pl.BlockSpec((8, 16), lambda i: (0, 0)),
      ],
      out_specs=[
          pl.BlockSpec((BN, 16), lambda i: (i, 0)),
          pl.BlockSpec((BN, 1), lambda i: (i, 0)),
      ],
      out_shape=[
          jax.ShapeDtypeStruct((NP, 16), jnp.float32),
          jax.ShapeDtypeStruct((NP, 1), jnp.float32),
      ],
  )(xp, deg_parts, w1p)


def _make_tc_layer(P, d_prev, d_out, sum_parts):
  def body(parts_ref, g_ref, dinv_ref, w_ref, out_ref):
    if sum_parts:
      agg = parts_ref[0] + parts_ref[1]
    else:
      agg = jnp.concatenate([parts_ref[p] for p in range(P)], axis=1)
    dinv = dinv_ref[...]
    x = jax.nn.relu(dinv * (agg + g_ref[...]))
    out_ref[...] = jnp.dot(_l2n(x), w_ref[...],
                           preferred_element_type=jnp.float32) * dinv

  def run(parts, g_prev, dinv, w):
    return pl.pallas_call(
        body,
        grid=(TC_GRID,),
        in_specs=[
            pl.BlockSpec((P, BN, 16), lambda i: (0, i, 0)),
            pl.BlockSpec((BN, d_prev), lambda i: (i, 0)),
            pl.BlockSpec((BN, 1), lambda i: (i, 0)),
            pl.BlockSpec((d_prev, d_out), lambda i: (0, 0)),
        ],
        out_specs=pl.BlockSpec((BN, d_out), lambda i: (i, 0)),
        out_shape=jax.ShapeDtypeStruct((NP, d_out), jnp.float32),
    )(parts, g_prev, dinv, w)

  return run


_tc_layer2 = _make_tc_layer(2, 16, 32, True)
_tc_layer3 = _make_tc_layer(2, 32, 64, False)
_tc_layer4 = _make_tc_layer(4, 64, 128, False)


def _tc_x5_body(parts_ref, g_ref, dinv_ref, out_ref):
  agg = jnp.concatenate([parts_ref[p] for p in range(8)], axis=1)
  out_ref[...] = jax.nn.relu(dinv_ref[...] * (agg + g_ref[...]))


def _tc_x5(parts, g4, dinv):
  return pl.pallas_call(
      _tc_x5_body,
      grid=(TC_GRID,),
      in_specs=[
          pl.BlockSpec((8, BN, 16), lambda i: (0, i, 0)),
          pl.BlockSpec((BN, 128), lambda i: (i, 0)),
          pl.BlockSpec((BN, 1), lambda i: (i, 0)),
      ],
      out_specs=pl.BlockSpec((BN, 128), lambda i: (i, 0)),
      out_shape=jax.ShapeDtypeStruct((NP, 128), jnp.float32),
  )(parts, g4, dinv)


def _tc_heads_body(tab_ref, w1_ref, b1_ref, w2_ref, b2_ref, out_ref):
  pooled = jnp.max(tab_ref[...], axis=0)
  pooled = jnp.maximum(pooled, 0.0)
  h = jax.nn.relu(jnp.dot(_l2n(pooled), w1_ref[...],
                          preferred_element_type=jnp.float32) + b1_ref[...])
  out_ref[...] = jax.nn.relu(jnp.dot(_l2n(h), w2_ref[...],
                                     preferred_element_type=jnp.float32)
                             + b2_ref[...])


def _tc_heads(tables, l1w, l1b, l2wp, l2bp):
  return pl.pallas_call(
      _tc_heads_body,
      out_shape=jax.ShapeDtypeStruct((G, 16), jnp.float32),
  )(tables, l1w, l1b, l2wp, l2bp)


def kernel(x, edge_index, batch, W1, W2, W3, W4, L1_W, L1_b, L2_W, L2_b):
  f32 = jnp.float32
  src = edge_index[0]
  dst = edge_index[1]

  pad = EP + 4 * EB - E
  pad_src = (jnp.arange(pad, dtype=jnp.int32) % 1024)
  pad_dst = NP + (jnp.arange(pad, dtype=jnp.int32) % 64)
  src_p = jnp.concatenate([src, pad_src])
  dst_p = jnp.concatenate([dst, pad_dst])

  xp = jnp.zeros((NP, 8), f32).at[:N, :2].set(x)
  batch_p = jnp.concatenate(
      [batch, jnp.full((NP - N,), G - 1, jnp.int32)])
  w1p = jnp.zeros((8, 16), f32).at[:2].set(W1)

  zeros2d = jnp.zeros((ZB2, 16), f32)
  zeros1d = jnp.zeros((ZB1,), f32)

  deg_parts = _deg_kernel(dst_p, zeros1d)

  g1, dinv = _tc1(xp, deg_parts.reshape(NSC, NP, 1), w1p)

  parts1 = _scatter1(g1, src_p, dst_p, zeros2d)
  g2 = _tc_layer2(parts1, g1, dinv, W2)

  parts2 = _scatter2(g2.reshape(NP * 2, 16), src_p, dst_p, zeros2d)
  g3 = _tc_layer3(parts2, g2, dinv, W3)

  parts3 = _scatter4(g3.reshape(NP * 4, 16), src_p, dst_p, zeros2d)
  g4 = _tc_layer4(parts3, g3, dinv, W4)

  parts4 = _scatter8(g4.reshape(NP * 8, 16), src_p, dst_p, zeros2d)
  x5 = _tc_x5(parts4, g4, dinv)

  tables = _pool_kernel(x5.reshape(NP * 128), batch_p)

  l2wp = jnp.zeros((64, 16), f32).at[:, :10].set(L2_W)
  l2bp = jnp.zeros((16,), f32).at[:10].set(L2_b)
  out = _tc_heads(tables.reshape(NSC * NTILE, G, G),
                  L1_W, L1_b.reshape(1, 64), l2wp, l2bp.reshape(1, 16))
  return out[:, :10]


# speedup vs baseline: 16.1507x; 1.1048x over previous
import functools

import jax
import jax.numpy as jnp
from jax import lax
from jax.experimental import pallas as pl
from jax.experimental.pallas import tpu as pltpu
from jax.experimental.pallas import tpu_sc as plsc

N = 100000
G = 128
E = 1600000

NSC = 2
NTILE = 16
NP = 100096
ACC_ROWS = 100224
TILE_N = NP // NTILE
ZTILE = ACC_ROWS // NTILE
POOL_N = NP // (NSC * NTILE)

EB = 128
EP = 1605632
NBLK = EP // EB
BLK_PER_TILE = NBLK // NTILE
BLK_PER_TILE_HALF = NBLK // (2 * NTILE)

_mesh = plsc.VectorSubcoreMesh(core_axis_name="c", subcore_axis_name="s")
_sc_params = pltpu.CompilerParams(use_tc_tiling_on_sc=False,
                                  needs_layout_passes=False)


ZB1 = 2088
ZB2 = 261


def _zero_acc_rows(zbuf, acc, t):
  nb = ZTILE // zbuf.shape[0]
  for j in range(nb):
    pltpu.sync_copy(zbuf, acc.at[pl.ds(t * ZTILE + j * zbuf.shape[0],
                                       zbuf.shape[0])])


def _sc_degree(dst_hbm, zeros_hbm, out_hbm, *scr):
  dbufs = [scr[3 * i:3 * i + 3] for i in range(4)]
  onesv, zbuf, stage, acc = scr[12:]
  c = lax.axis_index("c")
  t = lax.axis_index("s")
  pltpu.sync_copy(zeros_hbm, zbuf)
  _zero_acc_rows(zbuf, acc, t)
  for i in range(EB // 16):
    onesv[pl.ds(i * 16, 16)] = jnp.ones((16,), jnp.float32)
  plsc.subcore_barrier()

  base = (c * (NBLK // 2) + t * BLK_PER_TILE_HALF) * EB

  def start_e(off, db):
    pltpu.async_copy(dst_hbm.at[pl.ds(off, EB)], db[0], db[1])

  def wait_e(db):
    pltpu.make_async_copy(dst_hbm.at[pl.ds(0, EB)], db[0], db[1]).wait()

  def wait_scatter(db):
    pltpu.make_async_copy(onesv, acc.at[db[0]], db[2]).wait()

  start_e(base, dbufs[0])
  start_e(base + EB, dbufs[1])

  def body(s, carry):
    for ph in range(4):
      b = 4 * s + ph
      wait_e(dbufs[ph])
      pltpu.async_copy(onesv, acc.at[dbufs[ph][0]], add=True,
                       sem=dbufs[ph][2])
      if ph < 2:
        @pl.when(s > 0)
        def _():
          wait_scatter(dbufs[(ph + 2) % 4])
      else:
        wait_scatter(dbufs[(ph + 2) % 4])
      start_e(base + (b + 2) * EB, dbufs[(ph + 2) % 4])
    return carry

  lax.fori_loop(0, BLK_PER_TILE_HALF // 4, body, 0)
  wait_scatter(dbufs[2])
  wait_scatter(dbufs[3])
  wait_e(dbufs[0])
  wait_e(dbufs[1])
  plsc.subcore_barrier()
  for j in range(2):
    sl_a = pl.ds(t * TILE_N + j * (TILE_N // 2), TILE_N // 2)
    sl_o = pl.ds(c * NP + t * TILE_N + j * (TILE_N // 2), TILE_N // 2)
    pltpu.sync_copy(acc.at[sl_a], stage)
    pltpu.sync_copy(stage, out_hbm.at[sl_o])


_deg_kernel = functools.partial(
    pl.kernel,
    out_type=jax.ShapeDtypeStruct((NSC * NP,), jnp.float32),
    mesh=_mesh,
    scratch_types=(
        [pltpu.VMEM((EB,), jnp.int32), pltpu.SemaphoreType.DMA,
         pltpu.SemaphoreType.DMA] * 4
        + [
            pltpu.VMEM((EB,), jnp.float32),
            pltpu.VMEM((ZB1,), jnp.float32),
            pltpu.VMEM((TILE_N // 2,), jnp.float32),
            pltpu.VMEM_SHARED((ACC_ROWS,), jnp.float32),
        ]),
    compiler_params=_sc_params,
)(_sc_degree)


def _make_scatter(nchunk):
  nparts = max(2, nchunk)
  npass = max(1, nchunk // 2)

  NE = 8
  NR = 4

  def body(g_hbm, src_hbm, dst_hbm, zeros_hbm, out_hbm, *scr):
    ebufs = [scr[3 * i:3 * i + 3] for i in range(NE)]
    rbufs = [scr[3 * NE + 4 * i:3 * NE + 4 * i + 4] for i in range(NR)]
    zbuf, stage, acc = scr[3 * NE + 4 * NR:]
    c = lax.axis_index("c")
    t = lax.axis_index("s")

    def start_e(off, eb):
      srcv, dstv, semE = eb
      pltpu.async_copy(src_hbm.at[pl.ds(off, EB)], srcv, semE)
      pltpu.async_copy(dst_hbm.at[pl.ds(off, EB)], dstv, semE)

    def wait_e(eb):
      srcv, dstv, semE = eb
      pltpu.make_async_copy(src_hbm.at[pl.ds(0, EB)], srcv, semE).wait()
      pltpu.make_async_copy(dst_hbm.at[pl.ds(0, EB)], dstv, semE).wait()

    def start_g(eb, rb, chunk):
      srcv = eb[0]
      rows, gidxv, semG, _ = rb
      if nchunk == 1:
        idx_ref = srcv
      else:
        for i in range(EB // 16):
          sl = pl.ds(i * 16, 16)
          gidxv[sl] = srcv[sl] * nchunk + chunk
        idx_ref = gidxv
      pltpu.async_copy(g_hbm.at[idx_ref], rows, semG)

    def issue_scatter(eb, rb):
      dstv = eb[1]
      rows, _, semG, semS = rb
      pltpu.make_async_copy(g_hbm.at[dstv], rows, semG).wait()
      pltpu.async_copy(rows, acc.at[dstv], add=True, sem=semS)

    def wait_scatter(eb, rb):
      dstv = eb[1]
      rows, _, _, semS = rb
      pltpu.make_async_copy(rows, acc.at[dstv], semS).wait()

    pltpu.sync_copy(zeros_hbm, zbuf)
    for k in range(npass):
      _zero_acc_rows(zbuf, acc, t)
      plsc.subcore_barrier()
      if nchunk == 1:
        chunk = None
        part = c
        base = (c * (NBLK // 2) + t * BLK_PER_TILE_HALF) * EB
        nblocks = BLK_PER_TILE_HALF
      else:
        chunk = c + 2 * k
        part = chunk
        base = t * BLK_PER_TILE * EB
        nblocks = BLK_PER_TILE

      for j in range(4):
        start_e(base + j * EB, ebufs[j])

      def ebody(s, carry):
        for ph in range(NE):
          b = NE * s + ph
          if ph < 4:
            @pl.when(s > 0)
            def _():
              wait_scatter(ebufs[(ph + 4) % NE], rbufs[ph % NR])
          else:
            wait_scatter(ebufs[(ph + 4) % NE], rbufs[ph % NR])
          wait_e(ebufs[ph])
          start_g(ebufs[ph], rbufs[ph % NR], chunk)
          if ph < 2:
            @pl.when(s > 0)
            def _():
              issue_scatter(ebufs[(ph - 2) % NE], rbufs[(ph - 2) % NR])
          else:
            issue_scatter(ebufs[ph - 2], rbufs[(ph - 2) % NR])
          start_e(base + (b + 4) * EB, ebufs[(ph + 4) % NE])
        return carry

      lax.fori_loop(0, nblocks // NE, ebody, 0)
      issue_scatter(ebufs[NE - 2], rbufs[(NE - 2) % NR])
      issue_scatter(ebufs[NE - 1], rbufs[(NE - 1) % NR])
      for j in range(4):
        wait_scatter(ebufs[4 + j], rbufs[j])
      for j in range(4):
        wait_e(ebufs[j])
      plsc.subcore_barrier()
      for j in range(16):
        sl = pl.ds(t * TILE_N + j * (TILE_N // 16), TILE_N // 16)
        pltpu.sync_copy(acc.at[sl], stage)
        pltpu.sync_copy(stage, out_hbm.at[part, sl])
      plsc.subcore_barrier()

  escr = []
  for _ in range(NE):
    escr += [pltpu.VMEM((EB,), jnp.int32), pltpu.VMEM((EB,), jnp.int32),
             pltpu.SemaphoreType.DMA]
  for _ in range(NR):
    escr += [pltpu.VMEM((EB, 16), jnp.float32), pltpu.VMEM((EB,), jnp.int32),
             pltpu.SemaphoreType.DMA, pltpu.SemaphoreType.DMA]
  escr += [
      pltpu.VMEM((ZB2, 16), jnp.float32),
      pltpu.VMEM((TILE_N // 16, 16), jnp.float32),
      pltpu.VMEM_SHARED((ACC_ROWS, 16), jnp.float32),
  ]
  return pl.kernel(
      body,
      out_type=jax.ShapeDtypeStruct((nparts, NP, 16), jnp.float32),
      mesh=_mesh,
      scratch_types=escr,
      compiler_params=_sc_params,
  )


_scatter1 = _make_scatter(1)
_scatter2 = _make_scatter(2)
_scatter4 = _make_scatter(4)
_scatter8 = _make_scatter(8)


_POOL_SUB = 512


def _sc_pool(x5_hbm, batch_hbm, out_hbm, xv, bv, table):
  c = lax.axis_index("c")
  t = lax.axis_index("s")
  wid = t * NSC + c
  base = wid * POOL_N
  iota = lax.iota(jnp.int32, 16)
  neginf = jnp.full((16,), -jnp.inf, jnp.float32)

  def init(i, carry):
    plsc.store_scatter(table, [i * 16 + iota], neginf)
    return carry

  lax.fori_loop(0, (G * G) // 16, init, 0)

  nsub = -(-POOL_N // _POOL_SUB)
  for j in range(nsub):
    size = min(_POOL_SUB, POOL_N - j * _POOL_SUB)
    off = base + j * _POOL_SUB
    pltpu.sync_copy(x5_hbm.at[pl.ds(off * 128, size * 128)],
                    xv.at[pl.ds(0, size * 128)])
    pltpu.sync_copy(batch_hbm.at[pl.ds(off, size)], bv.at[pl.ds(0, size)])

    def rbody(r, carry):
      ridx = jnp.full((16,), r, jnp.int32)
      bid = plsc.load_gather(bv, [ridx])
      for c8 in range(8):
        xidx = r * 128 + c8 * 16 + iota
        tidx = bid * G + c8 * 16 + iota
        val = plsc.load_gather(xv, [xidx])
        cur = plsc.load_gather(table, [tidx])
        plsc.store_scatter(table, [tidx], jnp.maximum(cur, val))
      return carry

    lax.fori_loop(0, size, rbody, 0)

  pltpu.sync_copy(table, out_hbm.at[wid])


_pool_kernel = functools.partial(
    pl.kernel,
    out_type=jax.ShapeDtypeStruct((NSC * NTILE, G * G), jnp.float32),
    mesh=_mesh,
    scratch_types=[
        pltpu.VMEM((_POOL_SUB * 128,), jnp.float32),
        pltpu.VMEM((_POOL_SUB,), jnp.int32),
        pltpu.VMEM((G * G,), jnp.float32),
    ],
    compiler_params=_sc_params,
)(_sc_pool)


BN = 3128
TC_GRID = NP // BN


def _l2n(x):
  return x / (jnp.sqrt(jnp.sum(x * x, axis=1, keepdims=True)) + 1e-8)


def _tc1_body(x_ref, deg_ref, w_ref, g_ref, dinv_ref):
  deg = deg_ref[0] + deg_ref[1] + 1.0
  dinv = lax.rsqrt(deg)
  g = jnp.dot(_l2n(x_ref[...]), w_ref[...],
              preferred_element_type=jnp.float32) * dinv
  g_ref[...] = g
  dinv_ref[...] = dinv


def _tc1(xp, deg_parts, w1p):
  return pl.pallas_call(
      _tc1_body,
      grid=(TC_GRID,),
      in_specs=[
          pl.BlockSpec((BN, 8), lambda i: (i, 0)),
          pl.BlockSpec((2, BN, 1), lambda i: (0, i, 0)),
          pl.BlockSpec((8, 16), lambda i: (0, 0)),
      ],
      out_specs=[
          pl.BlockSpec((BN, 16), lambda i: (i, 0)),
          pl.BlockSpec((BN, 1), lambda i: (i, 0)),
      ],
      out_shape=[
          jax.ShapeDtypeStruct((NP, 16), jnp.float32),
          jax.ShapeDtypeStruct((NP, 1), jnp.float32),
      ],
  )(xp, deg_parts, w1p)


def _make_tc_layer(P, d_prev, d_out, sum_parts):
  def body(parts_ref, g_ref, dinv_ref, w_ref, out_ref):
    if sum_parts:
      agg = parts_ref[0] + parts_ref[1]
    else:
      agg = jnp.concatenate([parts_ref[p] for p in range(P)], axis=1)
    dinv = dinv_ref[...]
    x = jax.nn.relu(dinv * (agg + g_ref[...]))
    out_ref[...] = jnp.dot(_l2n(x), w_ref[...],
                           preferred_element_type=jnp.float32) * dinv

  def run(parts, g_prev, dinv, w):
    return pl.pallas_call(
        body,
        grid=(TC_GRID,),
        in_specs=[
            pl.BlockSpec((P, BN, 16), lambda i: (0, i, 0)),
            pl.BlockSpec((BN, d_prev), lambda i: (i, 0)),
            pl.BlockSpec((BN, 1), lambda i: (i, 0)),
            pl.BlockSpec((d_prev, d_out), lambda i: (0, 0)),
        ],
        out_specs=pl.BlockSpec((BN, d_out), lambda i: (i, 0)),
        out_shape=jax.ShapeDtypeStruct((NP, d_out), jnp.float32),
    )(parts, g_prev, dinv, w)

  return run


_tc_layer2 = _make_tc_layer(2, 16, 32, True)
_tc_layer3 = _make_tc_layer(2, 32, 64, False)
_tc_layer4 = _make_tc_layer(4, 64, 128, False)


def _tc_x5_body(parts_ref, g_ref, dinv_ref, out_ref):
  agg = jnp.concatenate([parts_ref[p] for p in range(8)], axis=1)
  out_ref[...] = jax.nn.relu(dinv_ref[...] * (agg + g_ref[...]))


def _tc_x5(parts, g4, dinv):
  return pl.pallas_call(
      _tc_x5_body,
      grid=(TC_GRID,),
      in_specs=[
          pl.BlockSpec((8, BN, 16), lambda i: (0, i, 0)),
          pl.BlockSpec((BN, 128), lambda i: (i, 0)),
          pl.BlockSpec((BN, 1), lambda i: (i, 0)),
      ],
      out_specs=pl.BlockSpec((BN, 128), lambda i: (i, 0)),
      out_shape=jax.ShapeDtypeStruct((NP, 128), jnp.float32),
  )(parts, g4, dinv)


def _tc_heads_body(tab_ref, w1_ref, b1_ref, w2_ref, b2_ref, out_ref):
  pooled = jnp.max(tab_ref[...], axis=0)
  pooled = jnp.maximum(pooled, 0.0)
  h = jax.nn.relu(jnp.dot(_l2n(pooled), w1_ref[...],
                          preferred_element_type=jnp.float32) + b1_ref[...])
  out_ref[...] = jax.nn.relu(jnp.dot(_l2n(h), w2_ref[...],
                                     preferred_element_type=jnp.float32)
                             + b2_ref[...])


def _tc_heads(tables, l1w, l1b, l2wp, l2bp):
  return pl.pallas_call(
      _tc_heads_body,
      out_shape=jax.ShapeDtypeStruct((G, 16), jnp.float32),
  )(tables, l1w, l1b, l2wp, l2bp)


def kernel(x, edge_index, batch, W1, W2, W3, W4, L1_W, L1_b, L2_W, L2_b):
  f32 = jnp.float32
  src = edge_index[0]
  dst = edge_index[1]

  pad = EP + 4 * EB - E
  pad_src = (jnp.arange(pad, dtype=jnp.int32) % 1024)
  pad_dst = NP + (jnp.arange(pad, dtype=jnp.int32) % 64)
  src_p = jnp.concatenate([src, pad_src])
  dst_p = jnp.concatenate([dst, pad_dst])

  xp = jnp.zeros((NP, 8), f32).at[:N, :2].set(x)
  batch_p = jnp.concatenate(
      [batch, jnp.full((NP - N,), G - 1, jnp.int32)])
  w1p = jnp.zeros((8, 16), f32).at[:2].set(W1)

  zeros2d = jnp.zeros((ZB2, 16), f32)
  zeros1d = jnp.zeros((ZB1,), f32)

  deg_parts = _deg_kernel(dst_p, zeros1d)

  g1, dinv = _tc1(xp, deg_parts.reshape(NSC, NP, 1), w1p)

  parts1 = _scatter1(g1, src_p, dst_p, zeros2d)
  g2 = _tc_layer2(parts1, g1, dinv, W2)

  parts2 = _scatter2(g2.reshape(NP * 2, 16), src_p, dst_p, zeros2d)
  g3 = _tc_layer3(parts2, g2, dinv, W3)

  parts3 = _scatter4(g3.reshape(NP * 4, 16), src_p, dst_p, zeros2d)
  g4 = _tc_layer4(parts3, g3, dinv, W4)

  parts4 = _scatter8(g4.reshape(NP * 8, 16), src_p, dst_p, zeros2d)
  x5 = _tc_x5(parts4, g4, dinv)

  tables = _pool_kernel(x5.reshape(NP * 128), batch_p)

  l2wp = jnp.zeros((64, 16), f32).at[:, :10].set(L2_W)
  l2bp = jnp.zeros((16,), f32).at[:10].set(L2_b)
  out = _tc_heads(tables.reshape(NSC * NTILE, G, G),
                  L1_W, L1_b.reshape(1, 64), l2wp, l2bp.reshape(1, 16))
  return out[:, :10]


# speedup vs baseline: 17.4472x; 1.0803x over previous
import functools

import jax
import jax.numpy as jnp
from jax import lax
from jax.experimental import pallas as pl
from jax.experimental.pallas import tpu as pltpu
from jax.experimental.pallas import tpu_sc as plsc

N = 100000
G = 128
E = 1600000

NSC = 2
NTILE = 16
NP = 100096
ACC_ROWS = 100224
TILE_N = NP // NTILE
ZTILE = ACC_ROWS // NTILE
POOL_N = NP // (NSC * NTILE)

EB = 128
EP = 1605632
NBLK = EP // EB
BLK_PER_TILE = NBLK // NTILE
BLK_PER_TILE_HALF = NBLK // (2 * NTILE)

_mesh = plsc.VectorSubcoreMesh(core_axis_name="c", subcore_axis_name="s")
_sc_params = pltpu.CompilerParams(use_tc_tiling_on_sc=False,
                                  needs_layout_passes=False)


ZB1 = 2088
ZB2 = 261


def _zero_acc_rows(zbuf, acc, t):
  nb = ZTILE // zbuf.shape[0]
  for j in range(nb):
    pltpu.sync_copy(zbuf, acc.at[pl.ds(t * ZTILE + j * zbuf.shape[0],
                                       zbuf.shape[0])])


def _sc_degree(dst_hbm, zeros_hbm, out_hbm, *scr):
  dbufs = [scr[3 * i:3 * i + 3] for i in range(4)]
  onesv, zbuf, stage, acc = scr[12:]
  c = lax.axis_index("c")
  t = lax.axis_index("s")
  pltpu.sync_copy(zeros_hbm, zbuf)
  _zero_acc_rows(zbuf, acc, t)
  for i in range(EB // 16):
    onesv[pl.ds(i * 16, 16)] = jnp.ones((16,), jnp.float32)
  plsc.subcore_barrier()

  base = (c * (NBLK // 2) + t * BLK_PER_TILE_HALF) * EB

  def start_e(off, db):
    pltpu.async_copy(dst_hbm.at[pl.ds(off, EB)], db[0], db[1])

  def wait_e(db):
    pltpu.make_async_copy(dst_hbm.at[pl.ds(0, EB)], db[0], db[1]).wait()

  def wait_scatter(db):
    pltpu.make_async_copy(onesv, acc.at[db[0]], db[2]).wait()

  start_e(base, dbufs[0])
  start_e(base + EB, dbufs[1])

  def body(s, carry):
    for ph in range(4):
      b = 4 * s + ph
      wait_e(dbufs[ph])
      pltpu.async_copy(onesv, acc.at[dbufs[ph][0]], add=True,
                       sem=dbufs[ph][2])
      if ph < 2:
        @pl.when(s > 0)
        def _():
          wait_scatter(dbufs[(ph + 2) % 4])
      else:
        wait_scatter(dbufs[(ph + 2) % 4])
      start_e(base + (b + 2) * EB, dbufs[(ph + 2) % 4])
    return carry

  lax.fori_loop(0, BLK_PER_TILE_HALF // 4, body, 0)
  wait_scatter(dbufs[2])
  wait_scatter(dbufs[3])
  wait_e(dbufs[0])
  wait_e(dbufs[1])
  plsc.subcore_barrier()
  for j in range(2):
    sl_a = pl.ds(t * TILE_N + j * (TILE_N // 2), TILE_N // 2)
    sl_o = pl.ds(c * NP + t * TILE_N + j * (TILE_N // 2), TILE_N // 2)
    pltpu.sync_copy(acc.at[sl_a], stage)
    pltpu.sync_copy(stage, out_hbm.at[sl_o])


_deg_kernel = functools.partial(
    pl.kernel,
    out_type=jax.ShapeDtypeStruct((NSC * NP,), jnp.float32),
    mesh=_mesh,
    scratch_types=(
        [pltpu.VMEM((EB,), jnp.int32), pltpu.SemaphoreType.DMA,
         pltpu.SemaphoreType.DMA] * 4
        + [
            pltpu.VMEM((EB,), jnp.float32),
            pltpu.VMEM((ZB1,), jnp.float32),
            pltpu.VMEM((TILE_N // 2,), jnp.float32),
            pltpu.VMEM_SHARED((ACC_ROWS,), jnp.float32),
        ]),
    compiler_params=_sc_params,
)(_sc_degree)


def _make_scatter(nchunk):
  nparts = max(2, nchunk)
  npass = max(1, nchunk // 2)

  NE = 8
  NR = 4

  def body(g_hbm, src_hbm, dst_hbm, zeros_hbm, out_hbm, *scr):
    ebufs = [scr[3 * i:3 * i + 3] for i in range(NE)]
    rbufs = [scr[3 * NE + 4 * i:3 * NE + 4 * i + 4] for i in range(NR)]
    zbuf, stage, acc = scr[3 * NE + 4 * NR:]
    c = lax.axis_index("c")
    t = lax.axis_index("s")

    def start_e(off, eb):
      srcv, dstv, semE = eb
      pltpu.async_copy(src_hbm.at[pl.ds(off, EB)], srcv, semE)
      pltpu.async_copy(dst_hbm.at[pl.ds(off, EB)], dstv, semE)

    def wait_e(eb):
      srcv, dstv, semE = eb
      pltpu.make_async_copy(src_hbm.at[pl.ds(0, EB)], srcv, semE).wait()
      pltpu.make_async_copy(dst_hbm.at[pl.ds(0, EB)], dstv, semE).wait()

    def start_g(eb, rb, chunk):
      srcv = eb[0]
      rows, gidxv, semG, _ = rb
      if nchunk == 1:
        idx_ref = srcv
      else:
        for i in range(EB // 16):
          sl = pl.ds(i * 16, 16)
          gidxv[sl] = srcv[sl] * nchunk + chunk
        idx_ref = gidxv
      pltpu.async_copy(g_hbm.at[idx_ref], rows, semG)

    def issue_scatter(eb, rb):
      dstv = eb[1]
      rows, _, semG, semS = rb
      pltpu.make_async_copy(g_hbm.at[dstv], rows, semG).wait()
      pltpu.async_copy(rows, acc.at[dstv], add=True, sem=semS)

    def wait_scatter(eb, rb):
      dstv = eb[1]
      rows, _, _, semS = rb
      pltpu.make_async_copy(rows, acc.at[dstv], semS).wait()

    pltpu.sync_copy(zeros_hbm, zbuf)
    for k in range(npass):
      _zero_acc_rows(zbuf, acc, t)
      plsc.subcore_barrier()
      if nchunk == 1:
        chunk = None
        part = c
        base = (c * (NBLK // 2) + t * BLK_PER_TILE_HALF) * EB
        nblocks = BLK_PER_TILE_HALF
      else:
        chunk = c + 2 * k
        part = chunk
        base = t * BLK_PER_TILE * EB
        nblocks = BLK_PER_TILE

      for j in range(4):
        start_e(base + j * EB, ebufs[j])

      def ebody(s, carry):
        for ph in range(NE):
          b = NE * s + ph
          if ph < 4:
            @pl.when(s > 0)
            def _():
              wait_scatter(ebufs[(ph + 4) % NE], rbufs[ph % NR])
          else:
            wait_scatter(ebufs[(ph + 4) % NE], rbufs[ph % NR])
          wait_e(ebufs[ph])
          start_g(ebufs[ph], rbufs[ph % NR], chunk)
          if ph < 2:
            @pl.when(s > 0)
            def _():
              issue_scatter(ebufs[(ph - 2) % NE], rbufs[(ph - 2) % NR])
          else:
            issue_scatter(ebufs[ph - 2], rbufs[(ph - 2) % NR])
          start_e(base + (b + 4) * EB, ebufs[(ph + 4) % NE])
        return carry

      lax.fori_loop(0, nblocks // NE, ebody, 0)
      issue_scatter(ebufs[NE - 2], rbufs[(NE - 2) % NR])
      issue_scatter(ebufs[NE - 1], rbufs[(NE - 1) % NR])
      for j in range(4):
        wait_scatter(ebufs[4 + j], rbufs[j])
      for j in range(4):
        wait_e(ebufs[j])
      plsc.subcore_barrier()
      for j in range(16):
        sl = pl.ds(t * TILE_N + j * (TILE_N // 16), TILE_N // 16)
        pltpu.sync_copy(acc.at[sl], stage)
        pltpu.sync_copy(stage, out_hbm.at[part, sl])
      plsc.subcore_barrier()

  escr = []
  for _ in range(NE):
    escr += [pltpu.VMEM((EB,), jnp.int32), pltpu.VMEM((EB,), jnp.int32),
             pltpu.SemaphoreType.DMA]
  for _ in range(NR):
    escr += [pltpu.VMEM((EB, 16), jnp.float32), pltpu.VMEM((EB,), jnp.int32),
             pltpu.SemaphoreType.DMA, pltpu.SemaphoreType.DMA]
  escr += [
      pltpu.VMEM((ZB2, 16), jnp.float32),
      pltpu.VMEM((TILE_N // 16, 16), jnp.float32),
      pltpu.VMEM_SHARED((ACC_ROWS, 16), jnp.float32),
  ]
  return pl.kernel(
      body,
      out_type=jax.ShapeDtypeStruct((nparts, NP, 16), jnp.float32),
      mesh=_mesh,
      scratch_types=escr,
      compiler_params=_sc_params,
  )


_scatter1 = _make_scatter(1)
_scatter2 = _make_scatter(2)
_scatter4 = _make_scatter(4)
_scatter8 = _make_scatter(8)


_POOL_SUB = 256


def _sc_pool(parts_hbm, g_hbm, dinv_hbm, batch_hbm, out_hbm,
             pv, gv, dv, bv, table):
  c = lax.axis_index("c")
  t = lax.axis_index("s")
  wid = t * NSC + c
  base = wid * POOL_N
  iota = lax.iota(jnp.int32, 16)
  neginf = jnp.full((16,), -jnp.inf, jnp.float32)

  def init(i, carry):
    plsc.store_scatter(table, [jnp.full((16,), i // 8, jnp.int32),
                               (i % 8) * 16 + iota], neginf)
    return carry

  lax.fori_loop(0, (G * G) // 16, init, 0)

  nsub = -(-POOL_N // _POOL_SUB)
  for j in range(nsub):
    size = min(_POOL_SUB, POOL_N - j * _POOL_SUB)
    off = base + j * _POOL_SUB
    for p in range(8):
      pltpu.sync_copy(parts_hbm.at[p, pl.ds(off, size)],
                      pv.at[p, pl.ds(0, size)])
    pltpu.sync_copy(g_hbm.at[pl.ds(off * 8, size * 8)],
                    gv.at[pl.ds(0, size * 8)])
    pltpu.sync_copy(dinv_hbm.at[pl.ds(off, size)], dv.at[pl.ds(0, size)])
    pltpu.sync_copy(batch_hbm.at[pl.ds(off, size)], bv.at[pl.ds(0, size)])

    def rbody(r, carry):
      ridx = jnp.full((16,), r, jnp.int32)
      bid = plsc.load_gather(bv, [ridx])
      dsp = plsc.load_gather(dv, [ridx])
      for c8 in range(8):
        cidx = jnp.full((16,), c8, jnp.int32)
        p4 = plsc.load_gather(pv, [cidx, ridx, iota])
        g4 = plsc.load_gather(gv, [ridx * 8 + c8, iota])
        val = jnp.maximum(dsp * (p4 + g4), 0.0)
        cur = plsc.load_gather(table, [bid, c8 * 16 + iota])
        plsc.store_scatter(table, [bid, c8 * 16 + iota],
                           jnp.maximum(cur, val))
      return carry

    lax.fori_loop(0, size, rbody, 0)

  pltpu.sync_copy(table, out_hbm.at[wid])


_pool_kernel = functools.partial(
    pl.kernel,
    out_type=jax.ShapeDtypeStruct((NSC * NTILE, G, G), jnp.float32),
    mesh=_mesh,
    scratch_types=[
        pltpu.VMEM((8, _POOL_SUB, 16), jnp.float32),
        pltpu.VMEM((_POOL_SUB * 8, 16), jnp.float32),
        pltpu.VMEM((_POOL_SUB,), jnp.float32),
        pltpu.VMEM((_POOL_SUB,), jnp.int32),
        pltpu.VMEM((G, G), jnp.float32),
    ],
    compiler_params=_sc_params,
)(_sc_pool)


BN = 3128
TC_GRID = NP // BN


def _l2n(x):
  return x / (jnp.sqrt(jnp.sum(x * x, axis=1, keepdims=True)) + 1e-8)


def _tc1_body(x_ref, deg_ref, w_ref, g_ref, dinv_ref):
  deg = deg_ref[0] + deg_ref[1] + 1.0
  dinv = lax.rsqrt(deg)
  g = jnp.dot(_l2n(x_ref[...]), w_ref[...],
              preferred_element_type=jnp.float32) * dinv
  g_ref[...] = g
  dinv_ref[...] = dinv


def _tc1(xp, deg_parts, w1p):
  return pl.pallas_call(
      _tc1_body,
      grid=(TC_GRID,),
      in_specs=[
          pl.BlockSpec((BN, 8), lambda i: (i, 0)),
          pl.BlockSpec((2, BN, 1), lambda i: (0, i, 0)),
          pl.BlockSpec((8, 16), lambda i: (0, 0)),
      ],
      out_specs=[
          pl.BlockSpec((BN, 16), lambda i: (i, 0)),
          pl.BlockSpec((BN, 1), lambda i: (i, 0)),
      ],
      out_shape=[
          jax.ShapeDtypeStruct((NP, 16), jnp.float32),
          jax.ShapeDtypeStruct((NP, 1), jnp.float32),
      ],
  )(xp, deg_parts, w1p)


def _make_tc_layer(P, d_prev, d_out, sum_parts):
  def body(parts_ref, g_ref, dinv_ref, w_ref, out_ref):
    if sum_parts:
      agg = parts_ref[0] + parts_ref[1]
    else:
      agg = jnp.concatenate([parts_ref[p] for p in range(P)], axis=1)
    dinv = dinv_ref[...]
    x = jax.nn.relu(dinv * (agg + g_ref[...]))
    out_ref[...] = jnp.dot(_l2n(x), w_ref[...],
                           preferred_element_type=jnp.float32) * dinv

  def run(parts, g_prev, dinv, w):
    return pl.pallas_call(
        body,
        grid=(TC_GRID,),
        in_specs=[
            pl.BlockSpec((P, BN, 16), lambda i: (0, i, 0)),
            pl.BlockSpec((BN, d_prev), lambda i: (i, 0)),
            pl.BlockSpec((BN, 1), lambda i: (i, 0)),
            pl.BlockSpec((d_prev, d_out), lambda i: (0, 0)),
        ],
        out_specs=pl.BlockSpec((BN, d_out), lambda i: (i, 0)),
        out_shape=jax.ShapeDtypeStruct((NP, d_out), jnp.float32),
    )(parts, g_prev, dinv, w)

  return run


_tc_layer2 = _make_tc_layer(2, 16, 32, True)
_tc_layer3 = _make_tc_layer(2, 32, 64, False)
_tc_layer4 = _make_tc_layer(4, 64, 128, False)


def _tc_heads_body(tab_ref, w1_ref, b1_ref, w2_ref, b2_ref, out_ref):
  pooled = jnp.max(tab_ref[...], axis=0)
  pooled = jnp.maximum(pooled, 0.0)
  h = jax.nn.relu(jnp.dot(_l2n(pooled), w1_ref[...],
                          preferred_element_type=jnp.float32) + b1_ref[...])
  out_ref[...] = jax.nn.relu(jnp.dot(_l2n(h), w2_ref[...],
                                     preferred_element_type=jnp.float32)
                             + b2_ref[...])


def _tc_heads(tables, l1w, l1b, l2wp, l2bp):
  return pl.pallas_call(
      _tc_heads_body,
      out_shape=jax.ShapeDtypeStruct((G, 16), jnp.float32),
  )(tables, l1w, l1b, l2wp, l2bp)


def kernel(x, edge_index, batch, W1, W2, W3, W4, L1_W, L1_b, L2_W, L2_b):
  f32 = jnp.float32
  src = edge_index[0]
  dst = edge_index[1]

  pad = EP + 4 * EB - E
  pad_src = (jnp.arange(pad, dtype=jnp.int32) % 1024)
  pad_dst = NP + (jnp.arange(pad, dtype=jnp.int32) % 64)
  src_p = jnp.concatenate([src, pad_src])
  dst_p = jnp.concatenate([dst, pad_dst])

  xp = jnp.zeros((NP, 8), f32).at[:N, :2].set(x)
  batch_p = jnp.concatenate(
      [batch, jnp.full((NP - N,), G - 1, jnp.int32)])
  w1p = jnp.zeros((8, 16), f32).at[:2].set(W1)

  zeros2d = jnp.zeros((ZB2, 16), f32)
  zeros1d = jnp.zeros((ZB1,), f32)

  deg_parts = _deg_kernel(dst_p, zeros1d)

  g1, dinv = _tc1(xp, deg_parts.reshape(NSC, NP, 1), w1p)

  parts1 = _scatter1(g1, src_p, dst_p, zeros2d)
  g2 = _tc_layer2(parts1, g1, dinv, W2)

  parts2 = _scatter2(g2.reshape(NP * 2, 16), src_p, dst_p, zeros2d)
  g3 = _tc_layer3(parts2, g2, dinv, W3)

  parts3 = _scatter4(g3.reshape(NP * 4, 16), src_p, dst_p, zeros2d)
  g4 = _tc_layer4(parts3, g3, dinv, W4)

  g4u = g4.reshape(NP * 8, 16)
  parts4 = _scatter8(g4u, src_p, dst_p, zeros2d)

  tables = _pool_kernel(parts4, g4u, dinv.reshape(NP), batch_p)

  l2wp = jnp.zeros((64, 16), f32).at[:, :10].set(L2_W)
  l2bp = jnp.zeros((16,), f32).at[:10].set(L2_b)
  out = _tc_heads(tables, L1_W, L1_b.reshape(1, 64), l2wp, l2bp.reshape(1, 16))
  return out[:, :10]


# speedup vs baseline: 17.4755x; 1.0016x over previous
import functools

import jax
import jax.numpy as jnp
from jax import lax
from jax.experimental import pallas as pl
from jax.experimental.pallas import tpu as pltpu
from jax.experimental.pallas import tpu_sc as plsc

N = 100000
G = 128
E = 1600000

NSC = 2
NTILE = 16
NP = 100096
ACC_ROWS = 100224
TILE_N = NP // NTILE
ZTILE = ACC_ROWS // NTILE
POOL_N = NP // (NSC * NTILE)

EB = 128
EP = 1605632
NBLK = EP // EB
BLK_PER_TILE = NBLK // NTILE
BLK_PER_TILE_HALF = NBLK // (2 * NTILE)

_mesh = plsc.VectorSubcoreMesh(core_axis_name="c", subcore_axis_name="s")
_sc_params = pltpu.CompilerParams(use_tc_tiling_on_sc=False,
                                  needs_layout_passes=False)


ZB1 = 2088
ZB2 = 261


def _zero_acc_rows(zbuf, acc, t):
  nb = ZTILE // zbuf.shape[0]
  for j in range(nb):
    pltpu.sync_copy(zbuf, acc.at[pl.ds(t * ZTILE + j * zbuf.shape[0],
                                       zbuf.shape[0])])


def _sc_degree(dst_hbm, zeros_hbm, out_hbm, *scr):
  dbufs = [scr[3 * i:3 * i + 3] for i in range(4)]
  onesv, zbuf, stage, acc = scr[12:]
  c = lax.axis_index("c")
  t = lax.axis_index("s")
  pltpu.sync_copy(zeros_hbm, zbuf)
  _zero_acc_rows(zbuf, acc, t)
  for i in range(EB // 16):
    onesv[pl.ds(i * 16, 16)] = jnp.ones((16,), jnp.float32)
  plsc.subcore_barrier()

  base = (c * (NBLK // 2) + t * BLK_PER_TILE_HALF) * EB

  def start_e(off, db):
    pltpu.async_copy(dst_hbm.at[pl.ds(off, EB)], db[0], db[1])

  def wait_e(db):
    pltpu.make_async_copy(dst_hbm.at[pl.ds(0, EB)], db[0], db[1]).wait()

  def wait_scatter(db):
    pltpu.make_async_copy(onesv, acc.at[db[0]], db[2]).wait()

  start_e(base, dbufs[0])
  start_e(base + EB, dbufs[1])

  def body(s, carry):
    for ph in range(4):
      b = 4 * s + ph
      wait_e(dbufs[ph])
      pltpu.async_copy(onesv, acc.at[dbufs[ph][0]], add=True,
                       sem=dbufs[ph][2])
      if ph < 2:
        @pl.when(s > 0)
        def _():
          wait_scatter(dbufs[(ph + 2) % 4])
      else:
        wait_scatter(dbufs[(ph + 2) % 4])
      start_e(base + (b + 2) * EB, dbufs[(ph + 2) % 4])
    return carry

  lax.fori_loop(0, BLK_PER_TILE_HALF // 4, body, 0)
  wait_scatter(dbufs[2])
  wait_scatter(dbufs[3])
  wait_e(dbufs[0])
  wait_e(dbufs[1])
  plsc.subcore_barrier()
  for j in range(2):
    sl_a = pl.ds(t * TILE_N + j * (TILE_N // 2), TILE_N // 2)
    sl_o = pl.ds(c * NP + t * TILE_N + j * (TILE_N // 2), TILE_N // 2)
    pltpu.sync_copy(acc.at[sl_a], stage)
    pltpu.sync_copy(stage, out_hbm.at[sl_o])


_deg_kernel = functools.partial(
    pl.kernel,
    out_type=jax.ShapeDtypeStruct((NSC * NP,), jnp.float32),
    mesh=_mesh,
    scratch_types=(
        [pltpu.VMEM((EB,), jnp.int32), pltpu.SemaphoreType.DMA,
         pltpu.SemaphoreType.DMA] * 4
        + [
            pltpu.VMEM((EB,), jnp.float32),
            pltpu.VMEM((ZB1,), jnp.float32),
            pltpu.VMEM((TILE_N // 2,), jnp.float32),
            pltpu.VMEM_SHARED((ACC_ROWS,), jnp.float32),
        ]),
    compiler_params=_sc_params,
)(_sc_degree)


def _make_scatter(nchunk):
  nparts = max(2, nchunk)
  npass = max(1, nchunk // 2)

  NE = 8
  NR = 4

  def body(g_hbm, ebl_hbm, zeros_hbm, out_hbm, *scr):
    ebufs = [scr[2 * i:2 * i + 2] for i in range(NE)]
    rbufs = [scr[2 * NE + 4 * i:2 * NE + 4 * i + 4] for i in range(NR)]
    zbuf, stage, acc = scr[2 * NE + 4 * NR:]
    c = lax.axis_index("c")
    t = lax.axis_index("s")

    def start_e(blk, eb):
      pltpu.async_copy(ebl_hbm.at[blk], eb[0], eb[1])

    def wait_e(eb):
      pltpu.make_async_copy(ebl_hbm.at[0], eb[0], eb[1]).wait()

    def start_g(eb, rb, chunk):
      ev = eb[0]
      rows, gidxv, semG, _ = rb
      if nchunk == 1:
        idx_ref = ev.at[0]
      else:
        for i in range(EB // 16):
          sl = pl.ds(i * 16, 16)
          gidxv[sl] = ev[0, sl] * nchunk + chunk
        idx_ref = gidxv
      pltpu.async_copy(g_hbm.at[idx_ref], rows, semG)

    def issue_scatter(eb, rb):
      dstv = eb[0].at[1]
      rows, _, semG, semS = rb
      pltpu.make_async_copy(g_hbm.at[dstv], rows, semG).wait()
      pltpu.async_copy(rows, acc.at[dstv], add=True, sem=semS)

    def wait_scatter(eb, rb):
      dstv = eb[0].at[1]
      rows, _, _, semS = rb
      pltpu.make_async_copy(rows, acc.at[dstv], semS).wait()

    pltpu.sync_copy(zeros_hbm, zbuf)
    for k in range(npass):
      _zero_acc_rows(zbuf, acc, t)
      plsc.subcore_barrier()
      if nchunk == 1:
        chunk = None
        part = c
        base = c * (NBLK // 2) + t * BLK_PER_TILE_HALF
        nblocks = BLK_PER_TILE_HALF
      else:
        chunk = c + 2 * k
        part = chunk
        base = t * BLK_PER_TILE
        nblocks = BLK_PER_TILE

      for j in range(4):
        start_e(base + j, ebufs[j])

      def ebody(s, carry):
        for ph in range(NE):
          b = NE * s + ph
          if ph < 4:
            @pl.when(s > 0)
            def _():
              wait_scatter(ebufs[(ph + 4) % NE], rbufs[ph % NR])
          else:
            wait_scatter(ebufs[(ph + 4) % NE], rbufs[ph % NR])
          wait_e(ebufs[ph])
          start_g(ebufs[ph], rbufs[ph % NR], chunk)
          if ph < 2:
            @pl.when(s > 0)
            def _():
              issue_scatter(ebufs[(ph - 2) % NE], rbufs[(ph - 2) % NR])
          else:
            issue_scatter(ebufs[ph - 2], rbufs[(ph - 2) % NR])
          start_e(base + b + 4, ebufs[(ph + 4) % NE])
        return carry

      lax.fori_loop(0, nblocks // NE, ebody, 0)
      issue_scatter(ebufs[NE - 2], rbufs[(NE - 2) % NR])
      issue_scatter(ebufs[NE - 1], rbufs[(NE - 1) % NR])
      for j in range(4):
        wait_scatter(ebufs[4 + j], rbufs[j])
      for j in range(4):
        wait_e(ebufs[j])
      plsc.subcore_barrier()
      for j in range(16):
        sl = pl.ds(t * TILE_N + j * (TILE_N // 16), TILE_N // 16)
        pltpu.sync_copy(acc.at[sl], stage)
        pltpu.sync_copy(stage, out_hbm.at[part, sl])
      plsc.subcore_barrier()

  escr = []
  for _ in range(NE):
    escr += [pltpu.VMEM((2, EB), jnp.int32), pltpu.SemaphoreType.DMA]
  for _ in range(NR):
    escr += [pltpu.VMEM((EB, 16), jnp.float32), pltpu.VMEM((EB,), jnp.int32),
             pltpu.SemaphoreType.DMA, pltpu.SemaphoreType.DMA]
  escr += [
      pltpu.VMEM((ZB2, 16), jnp.float32),
      pltpu.VMEM((TILE_N // 16, 16), jnp.float32),
      pltpu.VMEM_SHARED((ACC_ROWS, 16), jnp.float32),
  ]
  return pl.kernel(
      body,
      out_type=jax.ShapeDtypeStruct((nparts, NP, 16), jnp.float32),
      mesh=_mesh,
      scratch_types=escr,
      compiler_params=_sc_params,
  )


_scatter1 = _make_scatter(1)
_scatter2 = _make_scatter(2)
_scatter4 = _make_scatter(4)
_scatter8 = _make_scatter(8)


_POOL_SUB = 256


def _sc_pool(parts_hbm, g_hbm, dinv_hbm, batch_hbm, out_hbm,
             pv, gv, dv, bv, table):
  c = lax.axis_index("c")
  t = lax.axis_index("s")
  wid = t * NSC + c
  base = wid * POOL_N
  iota = lax.iota(jnp.int32, 16)
  neginf = jnp.full((16,), -jnp.inf, jnp.float32)

  def init(i, carry):
    plsc.store_scatter(table, [jnp.full((16,), i // 8, jnp.int32),
                               (i % 8) * 16 + iota], neginf)
    return carry

  lax.fori_loop(0, (G * G) // 16, init, 0)

  nsub = -(-POOL_N // _POOL_SUB)
  for j in range(nsub):
    size = min(_POOL_SUB, POOL_N - j * _POOL_SUB)
    off = base + j * _POOL_SUB
    for p in range(8):
      pltpu.sync_copy(parts_hbm.at[p, pl.ds(off, size)],
                      pv.at[p, pl.ds(0, size)])
    pltpu.sync_copy(g_hbm.at[pl.ds(off * 8, size * 8)],
                    gv.at[pl.ds(0, size * 8)])
    pltpu.sync_copy(dinv_hbm.at[pl.ds(off, size)], dv.at[pl.ds(0, size)])
    pltpu.sync_copy(batch_hbm.at[pl.ds(off, size)], bv.at[pl.ds(0, size)])

    def rbody(r, carry):
      ridx = jnp.full((16,), r, jnp.int32)
      bid = plsc.load_gather(bv, [ridx])
      dsp = plsc.load_gather(dv, [ridx])
      for c8 in range(8):
        cidx = jnp.full((16,), c8, jnp.int32)
        p4 = plsc.load_gather(pv, [cidx, ridx, iota])
        g4 = plsc.load_gather(gv, [ridx * 8 + c8, iota])
        val = jnp.maximum(dsp * (p4 + g4), 0.0)
        cur = plsc.load_gather(table, [bid, c8 * 16 + iota])
        plsc.store_scatter(table, [bid, c8 * 16 + iota],
                           jnp.maximum(cur, val))
      return carry

    lax.fori_loop(0, size, rbody, 0)

  pltpu.sync_copy(table, out_hbm.at[wid])


_pool_kernel = functools.partial(
    pl.kernel,
    out_type=jax.ShapeDtypeStruct((NSC * NTILE, G, G), jnp.float32),
    mesh=_mesh,
    scratch_types=[
        pltpu.VMEM((8, _POOL_SUB, 16), jnp.float32),
        pltpu.VMEM((_POOL_SUB * 8, 16), jnp.float32),
        pltpu.VMEM((_POOL_SUB,), jnp.float32),
        pltpu.VMEM((_POOL_SUB,), jnp.int32),
        pltpu.VMEM((G, G), jnp.float32),
    ],
    compiler_params=_sc_params,
)(_sc_pool)


BN = 3128
TC_GRID = NP // BN


def _l2n(x):
  return x / (jnp.sqrt(jnp.sum(x * x, axis=1, keepdims=True)) + 1e-8)


def _tc1_body(x_ref, deg_ref, w_ref, g_ref, dinv_ref):
  deg = deg_ref[0] + deg_ref[1] + 1.0
  dinv = lax.rsqrt(deg)
  g = jnp.dot(_l2n(x_ref[...]), w_ref[...],
              preferred_element_type=jnp.float32) * dinv
  g_ref[...] = g
  dinv_ref[...] = dinv


def _tc1(xp, deg_parts, w1p):
  return pl.pallas_call(
      _tc1_body,
      grid=(TC_GRID,),
      in_specs=[
          pl.BlockSpec((BN, 8), lambda i: (i, 0)),
          pl.BlockSpec((2, BN, 1), lambda i: (0, i, 0)),
          pl.BlockSpec((8, 16), lambda i: (0, 0)),
      ],
      out_specs=[
          pl.BlockSpec((BN, 16), lambda i: (i, 0)),
          pl.BlockSpec((BN, 1), lambda i: (i, 0)),
      ],
      out_shape=[
          jax.ShapeDtypeStruct((NP, 16), jnp.float32),
          jax.ShapeDtypeStruct((NP, 1), jnp.float32),
      ],
  )(xp, deg_parts, w1p)


def _make_tc_layer(P, d_prev, d_out, sum_parts):
  def body(parts_ref, g_ref, dinv_ref, w_ref, out_ref):
    if sum_parts:
      agg = parts_ref[0] + parts_ref[1]
    else:
      agg = jnp.concatenate([parts_ref[p] for p in range(P)], axis=1)
    dinv = dinv_ref[...]
    x = jax.nn.relu(dinv * (agg + g_ref[...]))
    out_ref[...] = jnp.dot(_l2n(x), w_ref[...],
                           preferred_element_type=jnp.float32) * dinv

  def run(parts, g_prev, dinv, w):
    return pl.pallas_call(
        body,
        grid=(TC_GRID,),
        in_specs=[
            pl.BlockSpec((P, BN, 16), lambda i: (0, i, 0)),
            pl.BlockSpec((BN, d_prev), lambda i: (i, 0)),
            pl.BlockSpec((BN, 1), lambda i: (i, 0)),
            pl.BlockSpec((d_prev, d_out), lambda i: (0, 0)),
        ],
        out_specs=pl.BlockSpec((BN, d_out), lambda i: (i, 0)),
        out_shape=jax.ShapeDtypeStruct((NP, d_out), jnp.float32),
    )(parts, g_prev, dinv, w)

  return run


_tc_layer2 = _make_tc_layer(2, 16, 32, True)
_tc_layer3 = _make_tc_layer(2, 32, 64, False)
_tc_layer4 = _make_tc_layer(4, 64, 128, False)


def _tc_heads_body(tab_ref, w1_ref, b1_ref, w2_ref, b2_ref, out_ref):
  pooled = jnp.max(tab_ref[...], axis=0)
  pooled = jnp.maximum(pooled, 0.0)
  h = jax.nn.relu(jnp.dot(_l2n(pooled), w1_ref[...],
                          preferred_element_type=jnp.float32) + b1_ref[...])
  out_ref[...] = jax.nn.relu(jnp.dot(_l2n(h), w2_ref[...],
                                     preferred_element_type=jnp.float32)
                             + b2_ref[...])


def _tc_heads(tables, l1w, l1b, l2wp, l2bp):
  return pl.pallas_call(
      _tc_heads_body,
      out_shape=jax.ShapeDtypeStruct((G, 16), jnp.float32),
  )(tables, l1w, l1b, l2wp, l2bp)


def kernel(x, edge_index, batch, W1, W2, W3, W4, L1_W, L1_b, L2_W, L2_b):
  f32 = jnp.float32
  src = edge_index[0]
  dst = edge_index[1]

  pad = EP + 4 * EB - E
  pad_src = (jnp.arange(pad, dtype=jnp.int32) % 1024)
  pad_dst = NP + (jnp.arange(pad, dtype=jnp.int32) % 64)
  src_p = jnp.concatenate([src, pad_src])
  dst_p = jnp.concatenate([dst, pad_dst])
  ebl = jnp.stack([src_p.reshape(-1, EB), dst_p.reshape(-1, EB)], axis=1)

  xp = jnp.zeros((NP, 8), f32).at[:N, :2].set(x)
  batch_p = jnp.concatenate(
      [batch, jnp.full((NP - N,), G - 1, jnp.int32)])
  w1p = jnp.zeros((8, 16), f32).at[:2].set(W1)

  zeros2d = jnp.zeros((ZB2, 16), f32)
  zeros1d = jnp.zeros((ZB1,), f32)

  deg_parts = _deg_kernel(dst_p, zeros1d)

  g1, dinv = _tc1(xp, deg_parts.reshape(NSC, NP, 1), w1p)

  parts1 = _scatter1(g1, ebl, zeros2d)
  g2 = _tc_layer2(parts1, g1, dinv, W2)

  parts2 = _scatter2(g2.reshape(NP * 2, 16), ebl, zeros2d)
  g3 = _tc_layer3(parts2, g2, dinv, W3)

  parts3 = _scatter4(g3.reshape(NP * 4, 16), ebl, zeros2d)
  g4 = _tc_layer4(parts3, g3, dinv, W4)

  g4u = g4.reshape(NP * 8, 16)
  parts4 = _scatter8(g4u, ebl, zeros2d)

  tables = _pool_kernel(parts4, g4u, dinv.reshape(NP), batch_p)

  l2wp = jnp.zeros((64, 16), f32).at[:, :10].set(L2_W)
  l2bp = jnp.zeros((16,), f32).at[:10].set(L2_b)
  out = _tc_heads(tables, L1_W, L1_b.reshape(1, 64), l2wp, l2bp.reshape(1, 16))
  return out[:, :10]
